# baseline (speedup 1.0000x reference)
import functools

import jax
import jax.numpy as jnp
from jax import lax
from jax.experimental import pallas as pl
from jax.experimental.pallas import tpu as pltpu
from jax.experimental.pallas import tpu_sc as plsc

DIM = 1024
NUM_EXPERTS = 8
TOP_K = 2
N_TOKENS = 4096
D_FF = 4 * DIM
CAPACITY = 1280
NE1 = NUM_EXPERTS + 1
FB = 1024
NF = D_FF // FB
TBLK = 1024
NW = 32
NSLOT = TOP_K * N_TOKENS
ZERO_ROW = NUM_EXPERTS * CAPACITY


def _gelu_exact(h):
    return h * 0.5 * (1.0 + lax.erf(h * 0.7071067811865476))


def _router_block(x_ref, wr_ref, probs_ref, i1_ref, i2_ref, w1_ref, w2_ref,
                  lz_ref):
    xb = x_ref[...]
    wr = wr_ref[...]
    logits = lax.dot_general(xb, wr, (((1,), (1,)), ((), ())),
                             preferred_element_type=jnp.float32)
    m = jnp.max(logits, axis=1, keepdims=True)
    ex = jnp.exp(logits - m)
    se = jnp.sum(ex, axis=1, keepdims=True)
    probs = ex / se
    probs_ref[...] = probs
    lz = m + jnp.log(se)
    lz_ref[...] = jnp.broadcast_to(lz, (TBLK, NUM_EXPERTS))
    cols = lax.broadcasted_iota(jnp.int32, (TBLK, NUM_EXPERTS), 1)
    m1 = jnp.max(probs, axis=1, keepdims=True)
    i1 = jnp.min(jnp.where(probs == m1, cols, NUM_EXPERTS), axis=1,
                 keepdims=True)
    pm = jnp.where(cols == i1, -jnp.inf, probs)
    m2 = jnp.max(pm, axis=1, keepdims=True)
    i2 = jnp.min(jnp.where(pm == m2, cols, NUM_EXPERTS), axis=1,
                 keepdims=True)
    s = jnp.maximum(m1 + m2, 1e-9)
    w1_ref[...] = jnp.broadcast_to(m1 / s, (TBLK, NUM_EXPERTS))
    w2_ref[...] = jnp.broadcast_to(m2 / s, (TBLK, NUM_EXPERTS))
    i1_ref[...] = jnp.broadcast_to(i1, (TBLK, NUM_EXPERTS))
    i2_ref[...] = jnp.broadcast_to(i2, (TBLK, NUM_EXPERTS))


def _router_call(x, w_router):
    n_blk = N_TOKENS // TBLK
    out8 = jax.ShapeDtypeStruct((N_TOKENS, NUM_EXPERTS), jnp.float32)
    out8i = jax.ShapeDtypeStruct((N_TOKENS, NUM_EXPERTS), jnp.int32)
    blk = pl.BlockSpec((TBLK, NUM_EXPERTS), lambda t: (t, 0))
    return pl.pallas_call(
        _router_block,
        grid=(n_blk,),
        in_specs=[
            pl.BlockSpec((TBLK, DIM), lambda t: (t, 0)),
            pl.BlockSpec((NUM_EXPERTS, DIM), lambda t: (0, 0)),
        ],
        out_specs=[blk, blk, blk, blk, blk, blk],
        out_shape=[out8, out8i, out8i, out8, out8, out8],
    )(x, w_router)


def _ffn_block(xg_ref, w1_ref, b1_ref, w2_ref, b2_ref, g_ref, y_ref, acc_ref):
    e = pl.program_id(0)
    f = pl.program_id(1)

    @pl.when(f == 0)
    def _init():
        acc_ref[...] = jnp.zeros_like(acc_ref)

    @pl.when(e < NUM_EXPERTS)
    def _compute():
        xb = xg_ref[...].astype(jnp.bfloat16)
        w1 = w1_ref[0].astype(jnp.bfloat16)
        h = lax.dot_general(xb, w1, (((1,), (1,)), ((), ())),
                            preferred_element_type=jnp.float32)
        h = _gelu_exact(h + b1_ref[0]).astype(jnp.bfloat16)
        w2 = w2_ref[0].astype(jnp.bfloat16)
        acc_ref[...] += lax.dot_general(h, w2, (((1,), (1,)), ((), ())),
                                        preferred_element_type=jnp.float32)

    @pl.when(f == NF - 1)
    def _emit():
        y_ref[...] = (acc_ref[...] + b2_ref[0]) * g_ref[:, 0:1]


def _ffn_call(xg, w1, b1, w2, b2, g2d):
    emap = lambda e: jnp.minimum(e, NUM_EXPERTS - 1)
    return pl.pallas_call(
        _ffn_block,
        grid=(NE1, NF),
        in_specs=[
            pl.BlockSpec((CAPACITY, DIM), lambda e, f: (emap(e), 0)),
            pl.BlockSpec((1, FB, DIM), lambda e, f: (emap(e), f, 0)),
            pl.BlockSpec((1, 1, FB), lambda e, f: (emap(e), 0, f)),
            pl.BlockSpec((1, DIM, FB), lambda e, f: (emap(e), 0, f)),
            pl.BlockSpec((1, 1, DIM), lambda e, f: (emap(e), 0, 0)),
            pl.BlockSpec((CAPACITY, 128), lambda e, f: (e, 0)),
        ],
        out_specs=pl.BlockSpec((CAPACITY, DIM), lambda e, f: (e, 0)),
        out_shape=jax.ShapeDtypeStruct((NE1 * CAPACITY, DIM), jnp.float32),
        scratch_shapes=[pltpu.VMEM((CAPACITY, DIM), jnp.float32)],
    )(xg, w1, b1.reshape(NUM_EXPERTS, 1, D_FF),
      w2, b2.reshape(NUM_EXPERTS, 1, DIM), g2d)


def _sc_mesh():
    return plsc.VectorSubcoreMesh(core_axis_name="c", subcore_axis_name="s",
                                  num_cores=2)


GCH = 40
GRPW = NUM_EXPERTS * CAPACITY // NW
GNCH = GRPW // GCH


def _sc_gather_body(x_hbm, tok_hbm, out_hbm, idx_v, rows0, rows1, sem0, sem1):
    wid = lax.axis_index("s") * 2 + lax.axis_index("c")
    base = wid * GRPW
    pltpu.sync_copy(tok_hbm.at[pl.ds(base, GRPW)], idx_v)
    bufs = (rows0, rows1)
    sems = (sem0, sem1)

    def start(i):
        return pltpu.async_copy(
            x_hbm.at[idx_v.at[pl.ds(i * GCH, GCH)]], bufs[i % 2], sems[i % 2])

    cps = [start(0), start(1)]
    for i in range(GNCH):
        cps[i].wait()
        pltpu.sync_copy(bufs[i % 2], out_hbm.at[pl.ds(base + i * GCH, GCH)])
        if i + 2 < GNCH:
            cps.append(start(i + 2))


def _sc_gather_rows(x, tok_dst):
    fn = functools.partial(
        pl.kernel, mesh=_sc_mesh(),
        out_type=jax.ShapeDtypeStruct((NUM_EXPERTS * CAPACITY, DIM),
                                      jnp.float32),
        scratch_types=[
            pltpu.VMEM((GRPW,), jnp.int32),
            pltpu.VMEM((GCH, DIM), jnp.float32),
            pltpu.VMEM((GCH, DIM), jnp.float32),
            pltpu.SemaphoreType.DMA,
            pltpu.SemaphoreType.DMA,
        ],
    )(_sc_gather_body)
    return fn(x, tok_dst)


CCH = 16
CRPW = N_TOKENS // NW
CNCH = CRPW // CCH


def _sc_combine_body(y_hbm, p0_hbm, p1_hbm, out_hbm, i0_v, i1_v,
                     ra0, rb0, ra1, rb1, sa0, sb0, sa1, sb1):
    wid = lax.axis_index("s") * 2 + lax.axis_index("c")
    base = wid * CRPW
    pltpu.sync_copy(p0_hbm.at[pl.ds(base, CRPW)], i0_v)
    pltpu.sync_copy(p1_hbm.at[pl.ds(base, CRPW)], i1_v)
    ra = (ra0, ra1)
    rb = (rb0, rb1)
    sa = (sa0, sa1)
    sb = (sb0, sb1)

    def start(i):
        k = i % 2
        sl = pl.ds(i * CCH, CCH)
        return (pltpu.async_copy(y_hbm.at[i0_v.at[sl]], ra[k], sa[k]),
                pltpu.async_copy(y_hbm.at[i1_v.at[sl]], rb[k], sb[k]))

    cps = [start(0), start(1)]
    for i in range(CNCH):
        k = i % 2
        cps[i][0].wait()
        cps[i][1].wait()

        def add_row(r, _, _k=k):
            for j in range(DIM // 16):
                sl = pl.ds(j * 16, 16)
                ra[_k][r, sl] = ra[_k][r, sl] + rb[_k][r, sl]
            return 0

        lax.fori_loop(0, CCH, add_row, 0)
        pltpu.sync_copy(ra[k], out_hbm.at[pl.ds(base + i * CCH, CCH)])
        if i + 2 < CNCH:
            cps.append(start(i + 2))


def _sc_combine(y, p0, p1):
    fn = functools.partial(
        pl.kernel, mesh=_sc_mesh(),
        out_type=jax.ShapeDtypeStruct((N_TOKENS, DIM), jnp.float32),
        scratch_types=[
            pltpu.VMEM((CRPW,), jnp.int32),
            pltpu.VMEM((CRPW,), jnp.int32),
            pltpu.VMEM((CCH, DIM), jnp.float32),
            pltpu.VMEM((CCH, DIM), jnp.float32),
            pltpu.VMEM((CCH, DIM), jnp.float32),
            pltpu.VMEM((CCH, DIM), jnp.float32),
            pltpu.SemaphoreType.DMA,
            pltpu.SemaphoreType.DMA,
            pltpu.SemaphoreType.DMA,
            pltpu.SemaphoreType.DMA,
        ],
    )(_sc_combine_body)
    return fn(y, p0, p1)


def kernel(x, W_router, W1, b1, W2, b2):
    probs, i1b, i2b, w1b, w2b, lzb = _router_call(x, W_router)
    i1 = i1b[:, 0]
    i2 = i2b[:, 0]
    g1 = w1b[:, 0]
    g2 = w2b[:, 0]
    logz = lzb[:, 0]

    iflat = jnp.stack([i1, i2], axis=1).reshape(-1)
    wflat = jnp.stack([g1, g2], axis=1).reshape(-1)
    slot = jnp.arange(NSLOT, dtype=jnp.int32)
    sk, snw, ss = lax.sort((iflat, -wflat, slot), num_keys=2, is_stable=True)

    counts = jnp.bincount(iflat, length=NUM_EXPERTS)
    start = jnp.concatenate(
        [jnp.zeros((1,), counts.dtype), jnp.cumsum(counts)[:-1]])
    posr = slot - start[sk].astype(jnp.int32)
    keep = posr < CAPACITY
    dst = sk * CAPACITY + posr

    g_pad = jnp.where(keep, dst, NE1 * CAPACITY)
    g_dst = jnp.zeros((NE1 * CAPACITY + 1,), jnp.float32
                      ).at[g_pad].set(-snw)[:NE1 * CAPACITY]
    t_pad = jnp.where(keep, dst, NUM_EXPERTS * CAPACITY)
    tok_dst = jnp.zeros((NUM_EXPERTS * CAPACITY + 1,), jnp.int32
                        ).at[t_pad].set(ss // TOP_K)[:NUM_EXPERTS * CAPACITY]
    pos_slot = jnp.zeros((NSLOT,), jnp.int32
                         ).at[ss].set(jnp.where(keep, dst, ZERO_ROW))
    pos = pos_slot.reshape(N_TOKENS, TOP_K)

    xg = _sc_gather_rows(x, tok_dst)
    g2d = jnp.broadcast_to(g_dst[:, None], (NE1 * CAPACITY, 128))
    y = _ffn_call(xg, W1, b1, W2, b2, g2d)
    out = _sc_combine(y, pos[:, 0], pos[:, 1])

    usage = jnp.mean(probs, axis=0)
    processed = jnp.minimum(counts, CAPACITY).astype(jnp.float32)
    denom = jnp.maximum(jnp.sum(processed), 1.0)
    load_balance = NUM_EXPERTS * jnp.sum(usage * (processed / denom))
    z_loss = jnp.mean(logz ** 2)
    aux = 0.01 * load_balance + 0.01 * z_loss
    return out, aux


# speedup vs baseline: 1.0254x; 1.0254x over previous
import functools

import jax
import jax.numpy as jnp
from jax import lax
from jax.experimental import pallas as pl
from jax.experimental.pallas import tpu as pltpu
from jax.experimental.pallas import tpu_sc as plsc

DIM = 1024
NUM_EXPERTS = 8
TOP_K = 2
N_TOKENS = 4096
D_FF = 4 * DIM
CAPACITY = 1280
NE1 = NUM_EXPERTS + 1
FB = 1024
NF = D_FF // FB
TBLK = 1024
NW = 32
NSLOT = TOP_K * N_TOKENS
ZERO_ROW = NUM_EXPERTS * CAPACITY


def _gelu_exact(h):
    return h * 0.5 * (1.0 + lax.erf(h * 0.7071067811865476))


def _router_block(x_ref, wr_ref, probs_ref, i1_ref, i2_ref, w1_ref, w2_ref,
                  lz_ref):
    xb = x_ref[...]
    wr = wr_ref[...]
    logits = lax.dot_general(xb, wr, (((1,), (1,)), ((), ())),
                             preferred_element_type=jnp.float32)
    m = jnp.max(logits, axis=1, keepdims=True)
    ex = jnp.exp(logits - m)
    se = jnp.sum(ex, axis=1, keepdims=True)
    probs = ex / se
    probs_ref[...] = probs
    lz = m + jnp.log(se)
    lz_ref[...] = jnp.broadcast_to(lz, (TBLK, NUM_EXPERTS))
    cols = lax.broadcasted_iota(jnp.int32, (TBLK, NUM_EXPERTS), 1)
    m1 = jnp.max(probs, axis=1, keepdims=True)
    i1 = jnp.min(jnp.where(probs == m1, cols, NUM_EXPERTS), axis=1,
                 keepdims=True)
    pm = jnp.where(cols == i1, -jnp.inf, probs)
    m2 = jnp.max(pm, axis=1, keepdims=True)
    i2 = jnp.min(jnp.where(pm == m2, cols, NUM_EXPERTS), axis=1,
                 keepdims=True)
    s = jnp.maximum(m1 + m2, 1e-9)
    w1_ref[...] = jnp.broadcast_to(m1 / s, (TBLK, NUM_EXPERTS))
    w2_ref[...] = jnp.broadcast_to(m2 / s, (TBLK, NUM_EXPERTS))
    i1_ref[...] = jnp.broadcast_to(i1, (TBLK, NUM_EXPERTS))
    i2_ref[...] = jnp.broadcast_to(i2, (TBLK, NUM_EXPERTS))


def _router_call(x, w_router):
    n_blk = N_TOKENS // TBLK
    out8 = jax.ShapeDtypeStruct((N_TOKENS, NUM_EXPERTS), jnp.float32)
    out8i = jax.ShapeDtypeStruct((N_TOKENS, NUM_EXPERTS), jnp.int32)
    blk = pl.BlockSpec((TBLK, NUM_EXPERTS), lambda t: (t, 0))
    return pl.pallas_call(
        _router_block,
        grid=(n_blk,),
        in_specs=[
            pl.BlockSpec((TBLK, DIM), lambda t: (t, 0)),
            pl.BlockSpec((NUM_EXPERTS, DIM), lambda t: (0, 0)),
        ],
        out_specs=[blk, blk, blk, blk, blk, blk],
        out_shape=[out8, out8i, out8i, out8, out8, out8],
    )(x, w_router)


EHALF = NUM_EXPERTS // 2


def _ffn_block_half(eoff, *refs):
    if eoff == 0:
        xg_ref, w1_ref, b1_ref, w2_ref, b2_ref, g_ref, y_ref, acc_ref = refs
    else:
        _, xg_ref, w1_ref, b1_ref, w2_ref, b2_ref, g_ref, y_ref, acc_ref = refs
    e = pl.program_id(0)
    f = pl.program_id(1)

    @pl.when(f == 0)
    def _init():
        acc_ref[...] = jnp.zeros_like(acc_ref)

    @pl.when(e + eoff < NUM_EXPERTS)
    def _compute():
        xb = xg_ref[...]
        h = lax.dot_general(xb, w1_ref[0], (((1,), (1,)), ((), ())),
                            preferred_element_type=jnp.float32)
        h = _gelu_exact(h + b1_ref[0])
        acc_ref[...] += lax.dot_general(h, w2_ref[0], (((1,), (1,)), ((), ())),
                                        preferred_element_type=jnp.float32)

    @pl.when(f == NF - 1)
    def _emit():
        y_ref[...] = (acc_ref[...] + b2_ref[0]) * g_ref[:, 0:1]


def _ffn_call_half(eoff, xg_half, w1, b1r, w2, b2r, g2d, y_prev=None):
    ne = EHALF if eoff == 0 else NE1 - EHALF
    wmap = lambda e: jnp.minimum(e + eoff, NUM_EXPERTS - 1)
    xmap = lambda e: jnp.minimum(e, EHALF - 1)
    in_specs = [
        pl.BlockSpec((CAPACITY, DIM), lambda e, f: (xmap(e), 0)),
        pl.BlockSpec((1, FB, DIM), lambda e, f: (wmap(e), f, 0)),
        pl.BlockSpec((1, 1, FB), lambda e, f: (wmap(e), 0, f)),
        pl.BlockSpec((1, DIM, FB), lambda e, f: (wmap(e), 0, f)),
        pl.BlockSpec((1, 1, DIM), lambda e, f: (wmap(e), 0, 0)),
        pl.BlockSpec((CAPACITY, 128), lambda e, f: (e + eoff, 0)),
    ]
    args = [xg_half, w1, b1r, w2, b2r, g2d]
    kwargs = {}
    if y_prev is not None:
        in_specs = [pl.BlockSpec(memory_space=pl.ANY)] + in_specs
        args = [y_prev] + args
        kwargs["input_output_aliases"] = {0: 0}
    return pl.pallas_call(
        functools.partial(_ffn_block_half, eoff),
        grid=(ne, NF),
        in_specs=in_specs,
        out_specs=pl.BlockSpec((CAPACITY, DIM), lambda e, f: (e + eoff, 0)),
        out_shape=jax.ShapeDtypeStruct((NE1 * CAPACITY, DIM), jnp.float32),
        scratch_shapes=[pltpu.VMEM((CAPACITY, DIM), jnp.float32)],
        **kwargs,
    )(*args)


def _sc_mesh():
    return plsc.VectorSubcoreMesh(core_axis_name="c", subcore_axis_name="s",
                                  num_cores=2)


GCH = 40
NGH = EHALF * CAPACITY
GRPW = NGH // NW
GNCH = GRPW // GCH


def _sc_gather_body(x_hbm, tok_hbm, out_hbm, idx_v, rows0, rows1, sem0, sem1):
    wid = lax.axis_index("s") * 2 + lax.axis_index("c")
    base = wid * GRPW
    pltpu.sync_copy(tok_hbm.at[pl.ds(base, GRPW)], idx_v)
    bufs = (rows0, rows1)
    sems = (sem0, sem1)

    def start(i):
        return pltpu.async_copy(
            x_hbm.at[idx_v.at[pl.ds(i * GCH, GCH)]], bufs[i % 2], sems[i % 2])

    cps = [start(0), start(1)]
    for i in range(GNCH):
        cps[i].wait()
        pltpu.sync_copy(bufs[i % 2], out_hbm.at[pl.ds(base + i * GCH, GCH)])
        if i + 2 < GNCH:
            cps.append(start(i + 2))


def _sc_gather_rows(x, tok_half):
    fn = functools.partial(
        pl.kernel, mesh=_sc_mesh(),
        out_type=jax.ShapeDtypeStruct((NGH, DIM), jnp.float32),
        scratch_types=[
            pltpu.VMEM((GRPW,), jnp.int32),
            pltpu.VMEM((GCH, DIM), jnp.float32),
            pltpu.VMEM((GCH, DIM), jnp.float32),
            pltpu.SemaphoreType.DMA,
            pltpu.SemaphoreType.DMA,
        ],
    )(_sc_gather_body)
    return fn(x, tok_half)


CCH = 16
CRPW = N_TOKENS // NW
CNCH = CRPW // CCH


def _sc_combine_body(y_hbm, p0_hbm, p1_hbm, out_hbm, i0_v, i1_v,
                     ra0, rb0, ra1, rb1, sa0, sb0, sa1, sb1):
    wid = lax.axis_index("s") * 2 + lax.axis_index("c")
    base = wid * CRPW
    pltpu.sync_copy(p0_hbm.at[pl.ds(base, CRPW)], i0_v)
    pltpu.sync_copy(p1_hbm.at[pl.ds(base, CRPW)], i1_v)
    ra = (ra0, ra1)
    rb = (rb0, rb1)
    sa = (sa0, sa1)
    sb = (sb0, sb1)

    def start(i):
        k = i % 2
        sl = pl.ds(i * CCH, CCH)
        return (pltpu.async_copy(y_hbm.at[i0_v.at[sl]], ra[k], sa[k]),
                pltpu.async_copy(y_hbm.at[i1_v.at[sl]], rb[k], sb[k]))

    cps = [start(0), start(1)]
    for i in range(CNCH):
        k = i % 2
        cps[i][0].wait()
        cps[i][1].wait()

        def add_row(r, _, _k=k):
            for j in range(DIM // 16):
                sl = pl.ds(j * 16, 16)
                ra[_k][r, sl] = ra[_k][r, sl] + rb[_k][r, sl]
            return 0

        lax.fori_loop(0, CCH, add_row, 0)
        pltpu.sync_copy(ra[k], out_hbm.at[pl.ds(base + i * CCH, CCH)])
        if i + 2 < CNCH:
            cps.append(start(i + 2))


def _sc_combine(y, p0, p1):
    fn = functools.partial(
        pl.kernel, mesh=_sc_mesh(),
        out_type=jax.ShapeDtypeStruct((N_TOKENS, DIM), jnp.float32),
        scratch_types=[
            pltpu.VMEM((CRPW,), jnp.int32),
            pltpu.VMEM((CRPW,), jnp.int32),
            pltpu.VMEM((CCH, DIM), jnp.float32),
            pltpu.VMEM((CCH, DIM), jnp.float32),
            pltpu.VMEM((CCH, DIM), jnp.float32),
            pltpu.VMEM((CCH, DIM), jnp.float32),
            pltpu.SemaphoreType.DMA,
            pltpu.SemaphoreType.DMA,
            pltpu.SemaphoreType.DMA,
            pltpu.SemaphoreType.DMA,
        ],
    )(_sc_combine_body)
    return fn(y, p0, p1)


def kernel(x, W_router, W1, b1, W2, b2):
    probs, i1b, i2b, w1b, w2b, lzb = _router_call(x, W_router)
    i1 = i1b[:, 0]
    i2 = i2b[:, 0]
    g1 = w1b[:, 0]
    g2 = w2b[:, 0]
    logz = lzb[:, 0]

    iflat = jnp.stack([i1, i2], axis=1).reshape(-1)
    wflat = jnp.stack([g1, g2], axis=1).reshape(-1)
    slot = jnp.arange(NSLOT, dtype=jnp.int32)
    sk, snw, ss = lax.sort((iflat, -wflat, slot), num_keys=2, is_stable=True)

    counts = jnp.bincount(iflat, length=NUM_EXPERTS)
    start = jnp.concatenate(
        [jnp.zeros((1,), counts.dtype), jnp.cumsum(counts)[:-1]])
    posr = slot - start[sk].astype(jnp.int32)
    keep = posr < CAPACITY
    dst = sk * CAPACITY + posr

    g_pad = jnp.where(keep, dst, NE1 * CAPACITY)
    g_dst = jnp.zeros((NE1 * CAPACITY + 1,), jnp.float32
                      ).at[g_pad].set(-snw)[:NE1 * CAPACITY]
    t_pad = jnp.where(keep, dst, NUM_EXPERTS * CAPACITY)
    tok_dst = jnp.zeros((NUM_EXPERTS * CAPACITY + 1,), jnp.int32
                        ).at[t_pad].set(ss // TOP_K)[:NUM_EXPERTS * CAPACITY]
    pos_slot = jnp.zeros((NSLOT,), jnp.int32
                         ).at[ss].set(jnp.where(keep, dst, ZERO_ROW))
    pos = pos_slot.reshape(N_TOKENS, TOP_K)

    g2d = jnp.broadcast_to(g_dst[:, None], (NE1 * CAPACITY, 128))
    b1r = b1.reshape(NUM_EXPERTS, 1, D_FF)
    b2r = b2.reshape(NUM_EXPERTS, 1, DIM)
    xg_a = _sc_gather_rows(x, tok_dst[:NGH])
    xg_b = _sc_gather_rows(x, tok_dst[NGH:])
    y_a = _ffn_call_half(0, xg_a, W1, b1r, W2, b2r, g2d)
    y = _ffn_call_half(EHALF, xg_b, W1, b1r, W2, b2r, g2d, y_prev=y_a)
    out = _sc_combine(y, pos[:, 0], pos[:, 1])

    usage = jnp.mean(probs, axis=0)
    processed = jnp.minimum(counts, CAPACITY).astype(jnp.float32)
    denom = jnp.maximum(jnp.sum(processed), 1.0)
    load_balance = NUM_EXPERTS * jnp.sum(usage * (processed / denom))
    z_loss = jnp.mean(logz ** 2)
    aux = 0.01 * load_balance + 0.01 * z_loss
    return out, aux
